# Initial kernel scaffold; baseline (speedup 1.0000x reference)
import jax
import jax.numpy as jnp
from jax.experimental import pallas as pl


def kernel(x, types, edge_types, edge_in_indexs, edge_out_indexs, edge_in_out_indexs, edge_in_out_head_tail, batch, embed_tables, hl_W, hl_b, edge_table, Win, Wout, Wq1, Wk1, Wv1, We, Wm2, Wq2, Wk2, Wv2, ht2, attn, W1, b1, W2, b2):
    raise NotImplementedError("write your pallas kernel here")


# speedup vs baseline: 2.6228x; 2.6228x over previous
import functools

import jax
import jax.numpy as jnp
import numpy as np
from jax import lax
from jax.experimental import pallas as pl
from jax.experimental.pallas import tpu as pltpu
from jax.experimental.pallas import tpu_sc as plsc

N = 50000; E = 10000; MI = 100000; MO = 100000
NT = 8; V = 20000; EV = 64; D = 128; L = 2
HE = 4; HN = 4; H = 4; G = 64
EPS = 1e-16
INV = 1.0 / np.sqrt(32.0)

NC, NS = 2, 16
NW = NC * NS
NP = 50176
Mp = 200704
BN = 512
BE = 400


def _elu(a):
    return jnp.where(a > 0, a, jnp.exp(a) - 1.0)


def _rep4():
    c = lax.broadcasted_iota(jnp.int32, (4, D), 1) // 32
    r = lax.broadcasted_iota(jnp.int32, (4, D), 0)
    return (c == r).astype(jnp.float32)


def _mesh():
    return plsc.VectorSubcoreMesh(core_axis_name="c", subcore_axis_name="s",
                                  num_cores=NC, num_subcores=NS)


def _sc_gather_call(table, idx):
    GB = 112
    nb = NP // (NW * GB)
    per_tec = nb * GB

    @functools.partial(
        pl.kernel, mesh=_mesh(),
        out_type=jax.ShapeDtypeStruct((NP, D), jnp.float32),
        scratch_types=[
            pltpu.VMEM((per_tec,), jnp.int32),
            pltpu.VMEM((GB, D), jnp.float32),
            pltpu.SemaphoreType.DMA,
        ])
    def k(table_h, idx_h, out_h, idxv, rowsv, sem):
        wid = lax.axis_index("s") * NC + lax.axis_index("c")
        base = pl.multiple_of(wid * per_tec, 8)
        pltpu.sync_copy(idx_h.at[pl.ds(base, per_tec)], idxv)

        def body(b, _):
            pltpu.async_copy(table_h.at[idxv.at[pl.ds(b * GB, GB)]],
                             rowsv, sem).wait()
            pltpu.sync_copy(rowsv,
                            out_h.at[pl.ds(pl.multiple_of(base + b * GB, 8),
                                           GB)])
            return 0
        lax.fori_loop(0, nb, body, 0)

    return k(table, idx)


BAT = 64


def _make_attn(Sq, U, CS, M, with_v, nch):
    B = BAT
    nbt = Mp // (NW * B)
    per_tec = nbt * B
    rows_per_tec = CS // NS
    nfull, rem = rows_per_tec // B, rows_per_tec % B

    scratch = [
        pltpu.VMEM((16,), jnp.int32),
        pltpu.VMEM((4 * B,), jnp.int32),
        pltpu.VMEM((B,), jnp.int32),
        pltpu.VMEM((B,), jnp.int32),
        pltpu.VMEM((2 * B, D), jnp.float32),
        pltpu.VMEM((B // 8, D), jnp.float32),
        pltpu.VMEM((B, D), jnp.float32),
        pltpu.VMEM_SHARED((CS + 16, D), jnp.float32),
        pltpu.SemaphoreType.DMA,
        pltpu.SemaphoreType.DMA,
    ]

    def body(idx2_h, qk_h, v_h, c0_h, num_h, warr_h,
             c0v, iv, vidxv, slocv, qkv, wst, sv, acc_sh, semg, semi):
        cid = lax.axis_index("c")
        sid = lax.axis_index("s")
        wid = sid * NC + cid
        lanes = lax.broadcasted_iota(jnp.int32, (16,), 0)
        zero16 = jnp.zeros((16,), jnp.float32)
        voff = jnp.full((16,), Sq, jnp.int32)
        r0 = pl.multiple_of(sid * rows_per_tec, 8)
        pltpu.sync_copy(c0_h, c0v)
        ibase = 2 * wid * per_tec

        def chunk(c, _):
            cvec = c0v[...] + c * CS

            def zrow(r, _):
                for j in range(D // 16):
                    sv[r, pl.ds(16 * j, 16)] = zero16
                return 0
            lax.fori_loop(0, B, zrow, 0)
            for f in range(nfull):
                pltpu.sync_copy(sv, acc_sh.at[pl.ds(r0 + f * B, B)])
            if rem:
                pltpu.sync_copy(sv.at[pl.ds(0, rem)],
                                acc_sh.at[pl.ds(r0 + nfull * B, rem)])

            @pl.when(sid == NS - 1)
            def _():
                pltpu.sync_copy(sv.at[pl.ds(0, 16)],
                                acc_sh.at[pl.ds(CS, 16)])
            plsc.subcore_barrier()

            pltpu.sync_copy(
                idx2_h.at[pl.ds(pl.multiple_of(ibase, 8), 2 * B)],
                iv.at[pl.ds(0, 2 * B)])

            def batch(b, _):
                d = (b % 2) * (2 * B)
                dn = ((b + 1) % 2) * (2 * B)

                @pl.when(b < nbt - 1)
                def _():
                    nxt = pl.multiple_of(ibase + 2 * (b + 1) * B, 8)
                    pltpu.async_copy(idx2_h.at[pl.ds(nxt, 2 * B)],
                                     iv.at[pl.ds(dn, 2 * B)], semi)

                off = wid * per_tec + b * B
                for j in range(B // 16):
                    sg = iv[pl.ds(d + 16 * j, 16)]
                    loc = sg - cvec
                    gm = off + 16 * j + lanes
                    valid = (loc >= 0) & (loc < CS) & (gm < M)
                    slocv[pl.ds(16 * j, 16)] = jnp.where(valid, loc, CS)
                    if with_v:
                        vidxv[pl.ds(16 * j, 16)] = (
                            iv[pl.ds(d + B + 16 * j, 16)] - voff)

                cg = pltpu.async_copy(qk_h.at[iv.at[pl.ds(d, 2 * B)]], qkv, semg)
                if with_v:
                    cvv = pltpu.async_copy(v_h.at[vidxv], sv, semg)
                cg.wait()
                if with_v:
                    cvv.wait()

                perms = [lanes ^ kk for kk in (1, 2, 4, 8)]

                def inc(r, _):
                    for g in range(8):
                        i = 8 * r + g
                        zr = zero16
                        for h in range(4):
                            qa = qkv[i, pl.ds(32 * h, 16)]
                            qb = qkv[i, pl.ds(32 * h + 16, 16)]
                            ka = qkv[B + i, pl.ds(32 * h, 16)]
                            kb = qkv[B + i, pl.ds(32 * h + 16, 16)]
                            t = qa * ka + qb * kb
                            for p in perms:
                                t = t + t.at[p].get(
                                    mode='promise_in_bounds')
                            wm = jnp.exp(t * INV)
                            sv[i, pl.ds(32 * h, 16)] = (
                                sv[i, pl.ds(32 * h, 16)] * wm)
                            sv[i, pl.ds(32 * h + 16, 16)] = (
                                sv[i, pl.ds(32 * h + 16, 16)] * wm)
                            zr = zr + jnp.where(lanes == h, wm, 0.0)
                        wst[r, pl.ds(16 * g, 16)] = zr
                    return 0
                lax.fori_loop(0, B // 8, inc, 0)
                pltpu.sync_copy(sv, acc_sh.at[slocv], add=True)

                @pl.when(c == 0)
                def _():
                    pltpu.sync_copy(
                        wst,
                        warr_h.at[pl.ds(pl.multiple_of(off // 8, 8),
                                        B // 8)])

                @pl.when(b < nbt - 1)
                def _():
                    pltpu.make_async_copy(idx2_h.at[pl.ds(0, 2 * B)],
                                          iv.at[pl.ds(dn, 2 * B)], semi).wait()
                return 0
            lax.fori_loop(0, nbt, batch, 0)
            plsc.subcore_barrier()

            co = c * CS
            for f in range(nfull):
                pltpu.sync_copy(
                    acc_sh.at[pl.ds(r0 + f * B, B)],
                    num_h.at[cid, pl.ds(pl.multiple_of(co + r0 + f * B, 8),
                                        B)])
            if rem:
                pltpu.sync_copy(
                    acc_sh.at[pl.ds(r0 + nfull * B, rem)],
                    num_h.at[cid,
                             pl.ds(pl.multiple_of(co + r0 + nfull * B, 8),
                                   rem)])
            plsc.subcore_barrier()
            return 0
        lax.fori_loop(0, nch, chunk, 0)

    return pl.kernel(
        body,
        mesh=_mesh(),
        out_type=(jax.ShapeDtypeStruct((NC, nch * CS, D), jnp.float32),
                  jax.ShapeDtypeStruct((Mp // 8, D), jnp.float32)),
        scratch_types=scratch)


def _make_attn_z(CS, M, nch):
    B = BAT
    nbt = Mp // (NW * B)
    per_tec = nbt * B
    rows_per_tec = CS // NS
    nfull, rem = rows_per_tec // B, rows_per_tec % B

    scratch = [
        pltpu.VMEM((16,), jnp.int32),
        pltpu.VMEM((4 * B,), jnp.int32),
        pltpu.VMEM((B,), jnp.int32),
        pltpu.VMEM((B // 8, D), jnp.float32),
        pltpu.VMEM((B, D), jnp.float32),
        pltpu.VMEM_SHARED((CS + 16, D), jnp.float32),
        pltpu.SemaphoreType.DMA,
    ]

    def body(idx2_h, warr_h, c0_h, num_h,
             c0v, iv, slocv, wst, sv, acc_sh, semi):
        cid = lax.axis_index("c")
        sid = lax.axis_index("s")
        wid = sid * NC + cid
        lanes = lax.broadcasted_iota(jnp.int32, (16,), 0)
        zero16 = jnp.zeros((16,), jnp.float32)
        r0 = pl.multiple_of(sid * rows_per_tec, 8)
        pltpu.sync_copy(c0_h, c0v)
        ibase = 2 * wid * per_tec

        def chunk(c, _):
            cvec = c0v[...] + c * CS

            def zrow(r, _):
                for j in range(D // 16):
                    sv[r, pl.ds(16 * j, 16)] = zero16
                return 0
            lax.fori_loop(0, B, zrow, 0)
            for f in range(nfull):
                pltpu.sync_copy(sv, acc_sh.at[pl.ds(r0 + f * B, B)])
            if rem:
                pltpu.sync_copy(sv.at[pl.ds(0, rem)],
                                acc_sh.at[pl.ds(r0 + nfull * B, rem)])

            @pl.when(sid == NS - 1)
            def _():
                pltpu.sync_copy(sv.at[pl.ds(0, 16)],
                                acc_sh.at[pl.ds(CS, 16)])
            plsc.subcore_barrier()

            pltpu.sync_copy(
                idx2_h.at[pl.ds(pl.multiple_of(ibase, 8), 2 * B)],
                iv.at[pl.ds(0, 2 * B)])

            def batch(b, _):
                d = (b % 2) * (2 * B)
                dn = ((b + 1) % 2) * (2 * B)

                @pl.when(b < nbt - 1)
                def _():
                    nxt = pl.multiple_of(ibase + 2 * (b + 1) * B, 8)
                    pltpu.async_copy(idx2_h.at[pl.ds(nxt, 2 * B)],
                                     iv.at[pl.ds(dn, 2 * B)], semi)

                off = wid * per_tec + b * B
                for j in range(B // 16):
                    sg = iv[pl.ds(d + 16 * j, 16)]
                    loc = sg - cvec
                    gm = off + 16 * j + lanes
                    valid = (loc >= 0) & (loc < CS) & (gm < M)
                    slocv[pl.ds(16 * j, 16)] = jnp.where(valid, loc, CS)

                pltpu.sync_copy(
                    warr_h.at[pl.ds(pl.multiple_of(off // 8, 8), B // 8)],
                    wst)

                def zb(r, _):
                    for g in range(8):
                        sv[8 * r + g, pl.ds(0, 16)] = wst[r,
                                                          pl.ds(16 * g, 16)]
                    return 0
                lax.fori_loop(0, B // 8, zb, 0)
                pltpu.sync_copy(sv, acc_sh.at[slocv], add=True)

                @pl.when(b < nbt - 1)
                def _():
                    pltpu.make_async_copy(
                        idx2_h.at[pl.ds(0, 2 * B)],
                        iv.at[pl.ds(dn, 2 * B)], semi).wait()
                return 0
            lax.fori_loop(0, nbt, batch, 0)
            plsc.subcore_barrier()

            co = c * CS
            for f in range(nfull):
                pltpu.sync_copy(
                    acc_sh.at[pl.ds(r0 + f * B, B)],
                    num_h.at[cid, pl.ds(pl.multiple_of(co + r0 + f * B, 8),
                                        B)])
            if rem:
                pltpu.sync_copy(
                    acc_sh.at[pl.ds(r0 + nfull * B, rem)],
                    num_h.at[cid,
                             pl.ds(pl.multiple_of(co + r0 + nfull * B, 8),
                                   rem)])
            plsc.subcore_barrier()
            return 0
        lax.fori_loop(0, nch, chunk, 0)

    return pl.kernel(
        body,
        mesh=_mesh(),
        out_type=jax.ShapeDtypeStruct((NC, nch * CS, D), jnp.float32),
        scratch_types=scratch)


CSC = 12544


def _sc_attn_call(idx2, qk, v, Sq, nch):
    c0v = jnp.zeros((16,), jnp.int32)
    vfn = _make_attn(Sq, v.shape[0], CSC, MI + MO, True, nch)
    zfn = _make_attn_z(CSC, MI + MO, nch)
    num, warr = vfn(idx2, qk, v, c0v)
    z = zfn(idx2, warr, c0v)
    return num, z


def _interleave_idx(seg, kidx):
    return jnp.stack([seg.reshape(Mp // BAT, BAT),
                      kidx.reshape(Mp // BAT, BAT)], axis=1).reshape(-1)


def _tc_weight_products(Wl, Wr):
    def body(a_ref, b_ref, o_ref):
        o_ref[0] = jnp.dot(a_ref[0], b_ref[0],
                           preferred_element_type=jnp.float32)
    return pl.pallas_call(
        body,
        grid=(16,),
        in_specs=[pl.BlockSpec((1, D, D), lambda i: (i, 0, 0)),
                  pl.BlockSpec((1, D, D), lambda i: (i, 0, 0))],
        out_specs=pl.BlockSpec((1, D, D), lambda i: (i, 0, 0)),
        out_shape=jax.ShapeDtypeStruct((16, D, D), jnp.float32),
    )(Wl, Wr)


def _tc_table2(embed_tables, hl_W, hl_b):
    BV = 400
    def body(e_ref, w_ref, b_ref, o_ref):
        o_ref[0] = jnp.dot(e_ref[0], w_ref[0],
                           preferred_element_type=jnp.float32) + b_ref[0]
    return pl.pallas_call(
        body,
        grid=(NT, V // BV),
        in_specs=[pl.BlockSpec((1, BV, D), lambda t, i: (t, i, 0)),
                  pl.BlockSpec((1, D, D), lambda t, i: (t, 0, 0)),
                  pl.BlockSpec((1, 1, D), lambda t, i: (t, 0, 0))],
        out_specs=pl.BlockSpec((1, BV, D), lambda t, i: (t, i, 0)),
        out_shape=jax.ShapeDtypeStruct((NT, V, D), jnp.float32),
    )(embed_tables, hl_W, hl_b.reshape(NT, 1, D))


def _tc_ea(edge_types3, edge_table):
    def body(t_ref, tab_ref, o_ref):
        et = t_ref[0, 0, :]
        oh = (et[:, None] ==
              lax.broadcasted_iota(jnp.int32, (BE, EV), 1)).astype(jnp.float32)
        o_ref[...] = jnp.dot(oh, tab_ref[...],
                             preferred_element_type=jnp.float32)
    return pl.pallas_call(
        body,
        grid=(E // BE,),
        in_specs=[pl.BlockSpec((1, 1, BE), lambda i: (i, 0, 0)),
                  pl.BlockSpec((EV, D), lambda i: (0, 0))],
        out_specs=pl.BlockSpec((BE, D), lambda i: (i, 0)),
        out_shape=jax.ShapeDtypeStruct((E, D), jnp.float32),
    )(edge_types3, edge_table)


def _tc_node_proj(x, W5):
    def body(x_ref, w_ref, o_ref):
        for j in range(5):
            o_ref[j] = jnp.dot(x_ref[...], w_ref[j],
                               preferred_element_type=jnp.float32)
    return pl.pallas_call(
        body,
        grid=(NP // BN,),
        in_specs=[pl.BlockSpec((BN, D), lambda i: (i, 0)),
                  pl.BlockSpec((5, D, D), lambda i: (0, 0, 0))],
        out_specs=pl.BlockSpec((5, BN, D), lambda i: (0, i, 0)),
        out_shape=jax.ShapeDtypeStruct((5, NP, D), jnp.float32),
    )(x, W5)


def _tc_edge_proj(ea, W2s):
    def body(e_ref, w_ref, q_ref, w_out_ref):
        q_ref[...] = jnp.dot(e_ref[...], w_ref[0],
                             preferred_element_type=jnp.float32)
        w_out_ref[...] = jnp.dot(e_ref[...], w_ref[1],
                                 preferred_element_type=jnp.float32)
    return pl.pallas_call(
        body,
        grid=(E // BE,),
        in_specs=[pl.BlockSpec((BE, D), lambda i: (i, 0)),
                  pl.BlockSpec((2, D, D), lambda i: (0, 0, 0))],
        out_specs=[pl.BlockSpec((BE, D), lambda i: (i, 0)),
                   pl.BlockSpec((BE, D), lambda i: (i, 0))],
        out_shape=[jax.ShapeDtypeStruct((E, D), jnp.float32),
                   jax.ShapeDtypeStruct((E, D), jnp.float32)],
    )(ea, W2s)


def _tc_edge_update(num, z, eawe, CW, crows):
    def body(n_ref, z_ref, ew_ref, cw_ref, cr_ref, e_ref, k2_ref, v2_ref):
        rep = _rep4()
        nm = n_ref[0] + n_ref[1]
        z4 = z_ref[0][:, :4] + z_ref[1][:, :4]
        den = jnp.dot(z4 + EPS, rep, preferred_element_type=jnp.float32)
        enew = _elu(nm / den + ew_ref[...])
        e_ref[...] = enew
        ek = jnp.dot(enew, cw_ref[0], preferred_element_type=jnp.float32)
        ev = jnp.dot(enew, cw_ref[1], preferred_element_type=jnp.float32)
        k2_ref[0] = ek + cr_ref[0][None, :]
        k2_ref[1] = ek + cr_ref[1][None, :]
        v2_ref[0] = ev + cr_ref[2][None, :]
        v2_ref[1] = ev + cr_ref[3][None, :]
    return pl.pallas_call(
        body,
        grid=(E // BE,),
        in_specs=[pl.BlockSpec((NC, BE, D), lambda i: (0, i, 0)),
                  pl.BlockSpec((NC, BE, D), lambda i: (0, i, 0)),
                  pl.BlockSpec((BE, D), lambda i: (i, 0)),
                  pl.BlockSpec((2, D, D), lambda i: (0, 0, 0)),
                  pl.BlockSpec((8, D), lambda i: (0, 0))],
        out_specs=[pl.BlockSpec((BE, D), lambda i: (i, 0)),
                   pl.BlockSpec((2, BE, D), lambda i: (0, i, 0)),
                   pl.BlockSpec((2, BE, D), lambda i: (0, i, 0))],
        out_shape=[jax.ShapeDtypeStruct((E, D), jnp.float32),
                   jax.ShapeDtypeStruct((2, E, D), jnp.float32),
                   jax.ShapeDtypeStruct((2, E, D), jnp.float32)],
    )(num, z, eawe, CW, crows)


def _tc_node_update(x, num2, z2):
    def body(x_ref, n_ref, z_ref, o_ref):
        rep = _rep4()
        nm = n_ref[0] + n_ref[1]
        z4 = z_ref[0][:, :4] + z_ref[1][:, :4]
        den = jnp.dot(z4 + EPS, rep, preferred_element_type=jnp.float32)
        o_ref[...] = x_ref[...] + _elu(nm / den)
    return pl.pallas_call(
        body,
        grid=(NP // BN,),
        in_specs=[pl.BlockSpec((BN, D), lambda i: (i, 0)),
                  pl.BlockSpec((NC, BN, D), lambda i: (0, i, 0)),
                  pl.BlockSpec((NC, BN, D), lambda i: (0, i, 0))],
        out_specs=pl.BlockSpec((BN, D), lambda i: (i, 0)),
        out_shape=jax.ShapeDtypeStruct((NP, D), jnp.float32),
    )(x, num2, z2)


def _tc_pool_mlp(x, batch3, attn_row, W1, b1, W2, b2):
    nblk = NP // BN

    def body(x_ref, b_ref, a_ref, w1_ref, b1_ref, w2_ref, b2_ref, o_ref,
             acc, accz):
        nb = pl.program_id(0)

        @pl.when(nb == 0)
        def _():
            acc[...] = jnp.zeros_like(acc)
            accz[...] = jnp.zeros_like(accz)

        rep = _rep4()
        rept = rep.T
        xb = x_ref[...]
        p = xb * a_ref[...]
        at4 = jnp.dot(p, rept, preferred_element_type=jnp.float32)
        w4 = jnp.exp(at4)
        wexp = jnp.dot(w4, rep, preferred_element_type=jnp.float32)
        bx = xb * wexp
        bb = b_ref[0, 0, :]
        oh = (bb[:, None] ==
              lax.broadcasted_iota(jnp.int32, (BN, G), 1)).astype(jnp.float32)
        acc[...] = acc[...] + lax.dot_general(
            oh, bx, (((0,), (0,)), ((), ())),
            preferred_element_type=jnp.float32)
        accz[...] = accz[...] + lax.dot_general(
            oh, w4, (((0,), (0,)), ((), ())),
            preferred_element_type=jnp.float32)

        @pl.when(nb == nblk - 1)
        def _():
            den = jnp.dot(accz[...] + EPS, rep,
                          preferred_element_type=jnp.float32)
            vpool = acc[...] / den
            h1 = _elu(jnp.dot(vpool, w1_ref[...],
                              preferred_element_type=jnp.float32)
                      + b1_ref[...])
            o_ref[...] = jnp.dot(h1, w2_ref[...],
                                 preferred_element_type=jnp.float32) \
                + b2_ref[...]

    return pl.pallas_call(
        body,
        grid=(nblk,),
        in_specs=[pl.BlockSpec((BN, D), lambda i: (i, 0)),
                  pl.BlockSpec((1, 1, BN), lambda i: (i, 0, 0)),
                  pl.BlockSpec((1, D), lambda i: (0, 0)),
                  pl.BlockSpec((D, 64), lambda i: (0, 0)),
                  pl.BlockSpec((1, 64), lambda i: (0, 0)),
                  pl.BlockSpec((64, 10), lambda i: (0, 0)),
                  pl.BlockSpec((1, 10), lambda i: (0, 0))],
        out_specs=pl.BlockSpec((G, 10), lambda i: (0, 0)),
        out_shape=jax.ShapeDtypeStruct((G, 10), jnp.float32),
        scratch_shapes=[pltpu.VMEM((G, D), jnp.float32),
                        pltpu.VMEM((G, 4), jnp.float32)],
    )(x, batch3, attn_row, W1, b1.reshape(1, 64), W2, b2.reshape(1, 10))


def kernel(x, types, edge_types, edge_in_indexs, edge_out_indexs,
           edge_in_out_indexs, edge_in_out_head_tail, batch,
           embed_tables, hl_W, hl_b, edge_table,
           Win, Wout, Wq1, Wk1, Wv1, We, Wm2, Wq2, Wk2, Wv2, ht2,
           attn, W1, b1, W2, b2):
    ein = edge_in_indexs.astype(jnp.int32)
    eout = edge_out_indexs.astype(jnp.int32)

    def pad2(r):
        return jnp.zeros((D, D), jnp.float32).at[:2].set(r)
    Wl = jnp.stack([Win[0], Win[0], Wout[0], Wout[0], Wm2[0], Wm2[0],
                    Win[1], Win[1], Wout[1], Wout[1], Wm2[1], Wm2[1],
                    pad2(ht2[0]), pad2(ht2[0]), pad2(ht2[1]), pad2(ht2[1])])
    Wr = jnp.stack([Wk1[0], Wv1[0], Wk1[0], Wv1[0], Wk2[0], Wv2[0],
                    Wk1[1], Wv1[1], Wk1[1], Wv1[1], Wk2[1], Wv2[1],
                    Wk2[0], Wv2[0], Wk2[1], Wv2[1]])
    WP = _tc_weight_products(Wl, Wr)

    table2 = _tc_table2(embed_tables, hl_W, hl_b).reshape(NT * V, D)
    gidx = (types.astype(jnp.int32) * V + x.astype(jnp.int32))
    gidx = jnp.concatenate([gidx, jnp.zeros((NP - N,), jnp.int32)])
    xc = _sc_gather_call(table2, gidx)

    ea = _tc_ea(edge_types.astype(jnp.int32).reshape(E // BE, 1, BE),
                edge_table)

    padm = jnp.zeros((Mp - MI - MO,), jnp.int32)
    seg1 = jnp.concatenate([ein[1], eout[1], padm])
    src1 = jnp.concatenate([ein[0], eout[0] + NP, padm])
    seg2 = jnp.concatenate([ein[0], eout[0], padm])
    src2 = jnp.concatenate([ein[1], eout[1] + E, padm])
    idx2_1 = _interleave_idx(seg1, src1 + E)
    idx2_2 = _interleave_idx(seg2, src2 + NP)

    for l in range(L):
        W5 = jnp.stack([WP[6 * l + 0], WP[6 * l + 2],
                        Wq2[l],
                        WP[6 * l + 1], WP[6 * l + 3]])
        P = _tc_node_proj(xc, W5)
        Kcat = P[0:2].reshape(2 * NP, D)
        Vcat = P[3:5].reshape(2 * NP, D)
        Q2 = P[2]
        Q1, EaWe = _tc_edge_proj(ea, jnp.stack([Wq1[l], We[l]]))

        QK1 = jnp.concatenate([Q1, Kcat])
        num1, z1 = _sc_attn_call(idx2_1, QK1, Vcat, E, 1)
        num1 = num1[:, :E]
        z1 = z1[:, :E]

        crows = jnp.concatenate([WP[12 + 2 * l][:2], WP[13 + 2 * l][:2],
                                 jnp.zeros((4, D), jnp.float32)])
        e_new, K2p, V2p = _tc_edge_update(
            num1, z1, EaWe, jnp.stack([WP[6 * l + 4], WP[6 * l + 5]]), crows)
        K2cat = K2p.reshape(2 * E, D)
        V2cat = V2p.reshape(2 * E, D)

        QK2 = jnp.concatenate([Q2, K2cat])
        num2, z2 = _sc_attn_call(idx2_2, QK2, V2cat, NP, NP // CSC)
        xc = _tc_node_update(xc, num2, z2)
        ea = e_new

    batch3 = jnp.concatenate([batch.astype(jnp.int32),
                              jnp.full((NP - N,), G, jnp.int32)])
    batch3 = batch3.reshape(NP // BN, 1, BN)
    return _tc_pool_mlp(xc, batch3, attn.reshape(1, D),
                        W1, b1, W2, b2)
